# baseline (speedup 1.0000x reference)
import functools

import jax
import jax.numpy as jnp
from jax.experimental import pallas as pl
from jax.experimental.pallas import tpu as pltpu

N = 32768
DIM = 1024
HID = 128
K = 32
CK = 4
BLK = 1024
NBLK = N // BLK
BROWS = BLK // 128
SROWS = N // 128

_NEG = -3.0e38
_BIG = 2**30


def _fused_kernel(x_blk_ref, w1_ref, b1_ref, w2_ref, x_hbm_ref, out_ref,
                  sc_ref, cv_ref, cr_ref, rows_ref, idx_ref, sems):
    i = pl.program_id(0)
    h = jnp.dot(x_blk_ref[...], w1_ref[...], preferred_element_type=jnp.float32)
    h = jnp.maximum(h + b1_ref[...], 0.0)
    s = jnp.dot(h, w2_ref[...], preferred_element_type=jnp.float32)
    s2 = s.reshape(BROWS, 128)
    sc_ref[pl.ds(i * BROWS, BROWS), :] = s2

    @pl.when(i == 0)
    def _init():
        cv_ref[...] = jnp.full((CK, 128), _NEG, jnp.float32)
        cr_ref[...] = jnp.full((CK, 128), _BIG, jnp.int32)

    rows_blk = jax.lax.broadcasted_iota(jnp.int32, (BROWS, 128), 0) + i * BROWS
    T = jnp.concatenate([cv_ref[...], s2], axis=0)
    R = jnp.concatenate([cr_ref[...], rows_blk], axis=0)
    nv, nr = [], []
    for _ in range(CK):
        m = jnp.max(T, axis=0, keepdims=True)
        r = jnp.min(jnp.where(T == m, R, _BIG), axis=0, keepdims=True)
        T = jnp.where((T == m) & (R == r), _NEG, T)
        nv.append(m)
        nr.append(r)
    cv_ref[...] = jnp.concatenate(nv, axis=0)
    cr_ref[...] = jnp.concatenate(nr, axis=0)

    @pl.when(i == NBLK - 1)
    def _topk_gather():
        lane = jax.lax.broadcasted_iota(jnp.int32, (CK, 128), 1)
        C0 = cv_ref[...]
        G0 = cr_ref[...] * 128 + lane
        pos = (jax.lax.broadcasted_iota(jnp.int32, (CK, 128), 0) * 128
               + jax.lax.broadcasted_iota(jnp.int32, (CK, 128), 1))
        x, g = C0, G0
        nn = CK * 128
        for kk in [2 << t for t in range(nn.bit_length() - 1)]:
            j = kk // 2
            while j >= 1:
                bitj = (pos & j) != 0
                flip = (pos & kk) != 0
                if j < 128:
                    px = jnp.where(bitj, jnp.roll(x, j, axis=1),
                                   jnp.roll(x, -j, axis=1))
                    pg = jnp.where(bitj, jnp.roll(g, j, axis=1),
                                   jnp.roll(g, -j, axis=1))
                else:
                    r = j // 128
                    px = jnp.where(bitj, jnp.roll(x, r, axis=0),
                                   jnp.roll(x, -r, axis=0))
                    pg = jnp.where(bitj, jnp.roll(g, r, axis=0),
                                   jnp.roll(g, -r, axis=0))
                gt = (x > px) | ((x == px) & (g < pg))
                take_self = gt ^ bitj ^ flip
                x = jnp.where(take_self, x, px)
                g = jnp.where(take_self, g, pg)
                j //= 2
        for k in range(K):
            idx_ref[0, k] = g[0, k]
        v32 = x[0:1, K - 1:K]
        g32 = g[0:1, K - 1:K]
        last_v = C0[CK - 1:CK, :]
        last_g = G0[CK - 1:CK, :]
        unsafe = (last_v > v32) | ((last_v == v32) & (last_g < g32))
        any_unsafe = jnp.max(unsafe.astype(jnp.int32))

        @pl.when(any_unsafe == 1)
        def _exact_fallback():
            s = sc_ref[...]
            row = jax.lax.broadcasted_iota(jnp.int32, (SROWS, 128), 0)
            fl = jax.lax.broadcasted_iota(jnp.int32, (SROWS, 128), 1)
            fg = row * 128 + fl
            for k in range(K):
                fm = jnp.max(s, axis=(0, 1), keepdims=True)
                fi = jnp.min(jnp.where(s == fm, fg, _BIG), axis=(0, 1),
                             keepdims=True)
                idx_ref[0, k] = fi[0, 0]
                s = jnp.where(fg == fi, _NEG, s)

        copies = []
        for k in range(K):
            cp = pltpu.make_async_copy(
                x_hbm_ref.at[pl.ds(idx_ref[0, k], 1), :],
                rows_ref.at[pl.ds(k, 1), :],
                sems.at[k],
            )
            cp.start()
            copies.append(cp)
        for cp in copies:
            cp.wait()
        out_ref[...] = jnp.sum(rows_ref[...], axis=0, keepdims=True) * (1.0 / K)


@functools.partial(jax.jit, static_argnames=("interpret",))
def kernel(x, W1, b1, W2, b2, interpret=False):
    pooled = pl.pallas_call(
        _fused_kernel,
        grid=(NBLK,),
        in_specs=[
            pl.BlockSpec((BLK, DIM), lambda i: (i, 0)),
            pl.BlockSpec((DIM, HID), lambda i: (0, 0)),
            pl.BlockSpec((1, HID), lambda i: (0, 0)),
            pl.BlockSpec((HID, 1), lambda i: (0, 0)),
            pl.BlockSpec(memory_space=pl.ANY),
        ],
        out_specs=pl.BlockSpec((1, DIM), lambda i: (0, 0)),
        out_shape=jax.ShapeDtypeStruct((1, DIM), jnp.float32),
        scratch_shapes=[
            pltpu.VMEM((SROWS, 128), jnp.float32),
            pltpu.VMEM((CK, 128), jnp.float32),
            pltpu.VMEM((CK, 128), jnp.int32),
            pltpu.VMEM((K, DIM), jnp.float32),
            pltpu.SMEM((1, K), jnp.int32),
            pltpu.SemaphoreType.DMA((K,)),
        ],
        interpret=interpret,
    )(x, W1, b1.reshape(1, HID), W2, x)
    return pooled.reshape(DIM)


# speedup vs baseline: 1.2200x; 1.2200x over previous
import functools

import jax
import jax.numpy as jnp
from jax.experimental import pallas as pl
from jax.experimental.pallas import tpu as pltpu

N = 32768
DIM = 1024
HID = 128
K = 32
CK = 4
BLK = 4096
NBLK = N // BLK
BROWS = BLK // 128
SROWS = N // 128

_NEG = -3.0e38
_BIG = 2**30


def _fused_kernel(x_blk_ref, w1_ref, b1_ref, w2_ref, x_hbm_ref, out_ref,
                  sc_ref, cv_ref, cr_ref, rows_ref, idx_ref, sems):
    i = pl.program_id(0)
    h = jnp.dot(x_blk_ref[...], w1_ref[...], preferred_element_type=jnp.float32)
    h = jnp.maximum(h + b1_ref[...], 0.0)
    s = jnp.dot(h, w2_ref[...], preferred_element_type=jnp.float32)
    s2 = s.reshape(BROWS, 128)
    sc_ref[pl.ds(i * BROWS, BROWS), :] = s2

    @pl.when(i == 0)
    def _init():
        cv_ref[...] = jnp.full((CK, 128), _NEG, jnp.float32)
        cr_ref[...] = jnp.full((CK, 128), _BIG, jnp.int32)

    rows_blk = jax.lax.broadcasted_iota(jnp.int32, (BROWS, 128), 0) + i * BROWS
    T = jnp.concatenate([cv_ref[...], s2], axis=0)
    R = jnp.concatenate([cr_ref[...], rows_blk], axis=0)
    nv, nr = [], []
    for _ in range(CK):
        m = jnp.max(T, axis=0, keepdims=True)
        r = jnp.min(jnp.where(T == m, R, _BIG), axis=0, keepdims=True)
        T = jnp.where((T == m) & (R == r), _NEG, T)
        nv.append(m)
        nr.append(r)
    cv_ref[...] = jnp.concatenate(nv, axis=0)
    cr_ref[...] = jnp.concatenate(nr, axis=0)

    @pl.when(i == NBLK - 1)
    def _topk_gather():
        lane = jax.lax.broadcasted_iota(jnp.int32, (CK, 128), 1)
        C0 = cv_ref[...]
        G0 = cr_ref[...] * 128 + lane
        pos = (jax.lax.broadcasted_iota(jnp.int32, (CK, 128), 0) * 128
               + jax.lax.broadcasted_iota(jnp.int32, (CK, 128), 1))
        x, g = C0, G0
        nn = CK * 128
        for kk in [2 << t for t in range(nn.bit_length() - 1)]:
            j = kk // 2
            while j >= 1:
                bitj = (pos & j) != 0
                flip = (pos & kk) != 0
                if j < 128:
                    px = jnp.where(bitj, jnp.roll(x, j, axis=1),
                                   jnp.roll(x, -j, axis=1))
                    pg = jnp.where(bitj, jnp.roll(g, j, axis=1),
                                   jnp.roll(g, -j, axis=1))
                else:
                    r = j // 128
                    px = jnp.where(bitj, jnp.roll(x, r, axis=0),
                                   jnp.roll(x, -r, axis=0))
                    pg = jnp.where(bitj, jnp.roll(g, r, axis=0),
                                   jnp.roll(g, -r, axis=0))
                gt = (x > px) | ((x == px) & (g < pg))
                take_self = gt ^ bitj ^ flip
                x = jnp.where(take_self, x, px)
                g = jnp.where(take_self, g, pg)
                j //= 2
        for k in range(K):
            idx_ref[0, k] = g[0, k]
        v32 = x[0:1, K - 1:K]
        g32 = g[0:1, K - 1:K]
        last_v = C0[CK - 1:CK, :]
        last_g = G0[CK - 1:CK, :]
        unsafe = (last_v > v32) | ((last_v == v32) & (last_g < g32))
        any_unsafe = jnp.max(unsafe.astype(jnp.int32))

        @pl.when(any_unsafe == 1)
        def _exact_fallback():
            s = sc_ref[...]
            row = jax.lax.broadcasted_iota(jnp.int32, (SROWS, 128), 0)
            fl = jax.lax.broadcasted_iota(jnp.int32, (SROWS, 128), 1)
            fg = row * 128 + fl
            for k in range(K):
                fm = jnp.max(s, axis=(0, 1), keepdims=True)
                fi = jnp.min(jnp.where(s == fm, fg, _BIG), axis=(0, 1),
                             keepdims=True)
                idx_ref[0, k] = fi[0, 0]
                s = jnp.where(fg == fi, _NEG, s)

        copies = []
        for k in range(K):
            cp = pltpu.make_async_copy(
                x_hbm_ref.at[pl.ds(idx_ref[0, k], 1), :],
                rows_ref.at[pl.ds(k, 1), :],
                sems.at[k],
            )
            cp.start()
            copies.append(cp)
        for cp in copies:
            cp.wait()
        out_ref[...] = jnp.sum(rows_ref[...], axis=0, keepdims=True) * (1.0 / K)


@functools.partial(jax.jit, static_argnames=("interpret",))
def kernel(x, W1, b1, W2, b2, interpret=False):
    pooled = pl.pallas_call(
        _fused_kernel,
        grid=(NBLK,),
        in_specs=[
            pl.BlockSpec((BLK, DIM), lambda i: (i, 0)),
            pl.BlockSpec((DIM, HID), lambda i: (0, 0)),
            pl.BlockSpec((1, HID), lambda i: (0, 0)),
            pl.BlockSpec((HID, 1), lambda i: (0, 0)),
            pl.BlockSpec(memory_space=pl.ANY),
        ],
        out_specs=pl.BlockSpec((1, DIM), lambda i: (0, 0)),
        out_shape=jax.ShapeDtypeStruct((1, DIM), jnp.float32),
        scratch_shapes=[
            pltpu.VMEM((SROWS, 128), jnp.float32),
            pltpu.VMEM((CK, 128), jnp.float32),
            pltpu.VMEM((CK, 128), jnp.int32),
            pltpu.VMEM((K, DIM), jnp.float32),
            pltpu.SMEM((1, K), jnp.int32),
            pltpu.SemaphoreType.DMA((K,)),
        ],
        interpret=interpret,
    )(x, W1, b1.reshape(1, HID), W2, x)
    return pooled.reshape(DIM)


# speedup vs baseline: 1.2221x; 1.0017x over previous
import jax
import jax.numpy as jnp
from jax.experimental import pallas as pl
from jax.experimental.pallas import tpu as pltpu

N = 32768
DIM = 1024
HID = 128
K = 32
CK = 4
BLK = 4096
NBLK = N // BLK
BROWS = BLK // 128
SROWS = N // 128

_NEG = -3.0e38
_BIG = 2**30


def _fused_kernel(x_blk_ref, w1_ref, b1_ref, w2_ref, x_hbm_ref, out_ref,
                  sc_ref, cv_ref, cr_ref, rows_ref, idx_ref, sems):
    i = pl.program_id(0)
    h = jnp.dot(x_blk_ref[...], w1_ref[...], preferred_element_type=jnp.float32)
    h = jnp.maximum(h + b1_ref[...], 0.0)
    s = jnp.dot(h, w2_ref[...], preferred_element_type=jnp.float32)
    s2 = s.reshape(BROWS, 128)
    sc_ref[pl.ds(i * BROWS, BROWS), :] = s2

    @pl.when(i == 0)
    def _init():
        cv_ref[...] = jnp.full((CK, 128), _NEG, jnp.float32)
        cr_ref[...] = jnp.full((CK, 128), _BIG, jnp.int32)

    rows_blk = jax.lax.broadcasted_iota(jnp.int32, (BROWS, 128), 0) + i * BROWS
    T = jnp.concatenate([cv_ref[...], s2], axis=0)
    R = jnp.concatenate([cr_ref[...], rows_blk], axis=0)
    nv, nr = [], []
    for _ in range(CK):
        m = jnp.max(T, axis=0, keepdims=True)
        r = jnp.min(jnp.where(T == m, R, _BIG), axis=0, keepdims=True)
        T = jnp.where((T == m) & (R == r), _NEG, T)
        nv.append(m)
        nr.append(r)
    cv_ref[...] = jnp.concatenate(nv, axis=0)
    cr_ref[...] = jnp.concatenate(nr, axis=0)

    @pl.when(i == NBLK - 1)
    def _topk_gather():
        lane = jax.lax.broadcasted_iota(jnp.int32, (CK, 128), 1)
        C0 = cv_ref[...]
        G0 = cr_ref[...] * 128 + lane
        pos = (jax.lax.broadcasted_iota(jnp.int32, (CK, 128), 0) * 128
               + jax.lax.broadcasted_iota(jnp.int32, (CK, 128), 1))
        x, g = C0, G0
        nn = CK * 128
        for kk in [2 << t for t in range(nn.bit_length() - 1)]:
            j = kk // 2
            while j >= 1:
                bitj = (pos & j) != 0
                flip = (pos & kk) != 0
                if j < 128:
                    px = jnp.where(bitj, jnp.roll(x, j, axis=1),
                                   jnp.roll(x, -j, axis=1))
                    pg = jnp.where(bitj, jnp.roll(g, j, axis=1),
                                   jnp.roll(g, -j, axis=1))
                else:
                    r = j // 128
                    px = jnp.where(bitj, jnp.roll(x, r, axis=0),
                                   jnp.roll(x, -r, axis=0))
                    pg = jnp.where(bitj, jnp.roll(g, r, axis=0),
                                   jnp.roll(g, -r, axis=0))
                gt = (x > px) | ((x == px) & (g < pg))
                take_self = gt ^ bitj ^ flip
                x = jnp.where(take_self, x, px)
                g = jnp.where(take_self, g, pg)
                j //= 2
        for k in range(K):
            idx_ref[0, k] = g[0, k]
        v32 = x[0:1, K - 1:K]
        g32 = g[0:1, K - 1:K]
        last_v = C0[CK - 1:CK, :]
        last_g = G0[CK - 1:CK, :]
        unsafe = (last_v > v32) | ((last_v == v32) & (last_g < g32))
        any_unsafe = jnp.max(unsafe.astype(jnp.int32))

        @pl.when(any_unsafe == 1)
        def _exact_fallback():
            s = sc_ref[...]
            row = jax.lax.broadcasted_iota(jnp.int32, (SROWS, 128), 0)
            fl = jax.lax.broadcasted_iota(jnp.int32, (SROWS, 128), 1)
            fg = row * 128 + fl
            for k in range(K):
                fm = jnp.max(s, axis=(0, 1), keepdims=True)
                fi = jnp.min(jnp.where(s == fm, fg, _BIG), axis=(0, 1),
                             keepdims=True)
                idx_ref[0, k] = fi[0, 0]
                s = jnp.where(fg == fi, _NEG, s)

        copies = []
        for k in range(K):
            cp = pltpu.make_async_copy(
                x_hbm_ref.at[pl.ds(idx_ref[0, k], 1), :],
                rows_ref.at[pl.ds(k, 1), :],
                sems.at[k],
            )
            cp.start()
            copies.append(cp)
        for cp in copies:
            cp.wait()
        out_ref[...] = jnp.sum(rows_ref[...], axis=0, keepdims=True) * (1.0 / K)


@jax.jit
def kernel(x, W1, b1, W2, b2):
    pooled = pl.pallas_call(
        _fused_kernel,
        grid=(NBLK,),
        in_specs=[
            pl.BlockSpec((BLK, DIM), lambda i: (i, 0)),
            pl.BlockSpec((DIM, HID), lambda i: (0, 0)),
            pl.BlockSpec((1, HID), lambda i: (0, 0)),
            pl.BlockSpec((HID, 1), lambda i: (0, 0)),
            pl.BlockSpec(memory_space=pl.ANY),
        ],
        out_specs=pl.BlockSpec((1, DIM), lambda i: (0, 0)),
        out_shape=jax.ShapeDtypeStruct((1, DIM), jnp.float32),
        scratch_shapes=[
            pltpu.VMEM((SROWS, 128), jnp.float32),
            pltpu.VMEM((CK, 128), jnp.float32),
            pltpu.VMEM((CK, 128), jnp.int32),
            pltpu.VMEM((K, DIM), jnp.float32),
            pltpu.SMEM((1, K), jnp.int32),
            pltpu.SemaphoreType.DMA((K,)),
        ],
    )(x, W1, b1.reshape(1, HID), W2, x)
    return pooled.reshape(DIM)
